# baseline (speedup 1.0000x reference)
import functools

import jax
import jax.numpy as jnp
from jax import lax
from jax.experimental import pallas as pl
from jax.experimental.pallas import tpu as pltpu
from jax.experimental.pallas import tpu_sc as plsc

_EMB_DIMS = (32, 32, 64, 32, 32)
_PACK_OFF = (0, 32, 64, 0, 32)
_TOTAL = 192
_MODEL = 768
_LG = 128
_BLK = 4096


def _gather_pack(x3, t0, t1, t2, t3, t4):
    n_groups = x3.shape[1]
    n_tok = n_groups * _LG
    tshapes = [(t.shape, t.dtype) for t in (t0, t1, t2, t3, t4)]
    info = plsc.get_sparse_core_info()
    nc = info.num_cores
    nw = nc * info.num_subcores
    g_per_w = n_groups // nw
    mesh = plsc.VectorSubcoreMesh(core_axis_name="c", subcore_axis_name="s")

    scratch = (
        [pltpu.VMEM((g_per_w, _LG), jnp.int32) for _ in range(5)]
        + [pltpu.VMEM((_LG, d), jnp.float32) for d in _EMB_DIMS]
        + [pltpu.VMEM((_LG, d), jnp.float32) for d in _EMB_DIMS]
        + [pltpu.VMEM((_LG, _LG), jnp.float32) for _ in range(4)]
        + [pltpu.VMEM_SHARED(sh, dt) for sh, dt in tshapes]
        + [pltpu.SemaphoreType.DMA,
           pltpu.SemaphoreType.DMA,
           pltpu.SemaphoreType.DMA]
    )

    @functools.partial(
        pl.kernel,
        mesh=mesh,
        out_type=jax.ShapeDtypeStruct((2 * n_tok, _LG), jnp.float32),
        scratch_types=scratch,
        compiler_params=pltpu.CompilerParams(use_tc_tiling_on_sc=False),
    )
    def k(x_ref, r0, r1, r2, r3, r4, h_ref,
          i0, i1, i2, i3, i4,
          a0, a1, a2, a3, a4,
          b0, b1, b2, b3, b4,
          cl0, cr0, cl1, cr1,
          s0, s1, s2, s3, s4,
          sg, ss0, ss1):
        tbls_hbm = (r0, r1, r2, r3, r4)
        tbls = (s0, s1, s2, s3, s4)
        idxs = (i0, i1, i2, i3, i4)
        rows = ((a0, a1, a2, a3, a4), (b0, b1, b2, b3, b4))
        cbufs = ((cl0, cr0), (cl1, cr1))
        ssems = (ss0, ss1)
        sid = lax.axis_index("s")
        wid = sid * nc + lax.axis_index("c")
        g0 = wid * g_per_w

        @pl.when(sid == 0)
        def _stage():
            for f in range(5):
                pltpu.sync_copy(tbls_hbm[f], tbls[f])

        def zrow(t, _):
            for j in range(4):
                z = jnp.zeros((16,), jnp.float32)
                cr0[t, pl.ds(64 + 16 * j, 16)] = z
                cr1[t, pl.ds(64 + 16 * j, 16)] = z
            return _
        lax.fori_loop(0, _LG, zrow, 0)

        for f in range(5):
            pltpu.sync_copy(x_ref.at[f, pl.ds(g0, g_per_w)], idxs[f])
        plsc.subcore_barrier()

        def fire(it):
            s = it % 2
            return [
                pltpu.async_copy(tbls[f].at[idxs[f].at[it]], rows[s][f], sg)
                for f in range(5)
            ]

        def repack(s):
            rr = rows[s]
            halves = cbufs[s]

            def body(t, _):
                for f in range(5):
                    half = halves[0] if f < 3 else halves[1]
                    off = _PACK_OFF[f]
                    for j in range(_EMB_DIMS[f] // 16):
                        half[t, pl.ds(off + 16 * j, 16)] = (
                            rr[f][t, pl.ds(16 * j, 16)])
                return _
            lax.fori_loop(0, _LG, body, 0)

        gathers = fire(0)
        pend_store = [None, None]
        for it in range(g_per_w):
            s = it % 2
            for cp in gathers:
                cp.wait()
            if it + 1 < g_per_w:
                gathers = fire(it + 1)
            if pend_store[s] is not None:
                for cp in pend_store[s]:
                    cp.wait()
            repack(s)
            row_l = (g0 + it) * _LG
            cl, cr = cbufs[s]
            pend_store[s] = [
                pltpu.async_copy(cl, h_ref.at[pl.ds(row_l, _LG)], ssems[s]),
                pltpu.async_copy(
                    cr, h_ref.at[pl.ds(n_tok + row_l, _LG)], ssems[s]),
            ]
        for s in range(2):
            if pend_store[s] is not None:
                for cp in pend_store[s]:
                    cp.wait()

    return k(x3, t0, t1, t2, t3, t4)


def _encode(h2, w_l, w_r, enc_b2, n_tok):
    nblk = n_tok // _BLK

    def body(hl_ref, hr_ref, wl_ref, wr_ref, b_ref, o_ref):
        o_ref[...] = (
            jnp.dot(hl_ref[...], wl_ref[...],
                    preferred_element_type=jnp.float32)
            + jnp.dot(hr_ref[...], wr_ref[...],
                      preferred_element_type=jnp.float32)
            + b_ref[...]
        )

    return pl.pallas_call(
        body,
        grid=(nblk,),
        in_specs=[
            pl.BlockSpec((_BLK, _LG), lambda i: (i, 0)),
            pl.BlockSpec((_BLK, _LG), lambda i, *, nb=nblk: (nb + i, 0)),
            pl.BlockSpec((_LG, _MODEL), lambda i: (0, 0)),
            pl.BlockSpec((_LG, _MODEL), lambda i: (0, 0)),
            pl.BlockSpec((1, _MODEL), lambda i: (0, 0)),
        ],
        out_specs=pl.BlockSpec((_BLK, _MODEL), lambda i: (i, 0)),
        out_shape=jax.ShapeDtypeStruct((n_tok, _MODEL), jnp.float32),
    )(h2, h2, w_l, w_r, enc_b2)


def kernel(x, table_0, table_1, table_2, table_3, table_4, enc_w, enc_b):
    b, s, f = x.shape
    n_tok = b * s
    xi = x.astype(jnp.int32)
    x3 = xi.reshape(n_tok // _LG, _LG, f).transpose(2, 0, 1)
    h2 = _gather_pack(x3, table_0, table_1, table_2, table_3, table_4)
    w_l = enc_w[:_LG]
    w_r = jnp.pad(enc_w[_LG:], ((0, 2 * _LG - _TOTAL), (0, 0)))
    out = _encode(h2, w_l, w_r, enc_b.reshape(1, _MODEL), n_tok)
    return out.reshape(b, s, _MODEL)


# speedup vs baseline: 1.3707x; 1.3707x over previous
import functools

import jax
import jax.numpy as jnp
from jax import lax
from jax.experimental import pallas as pl
from jax.experimental.pallas import tpu as pltpu
from jax.experimental.pallas import tpu_sc as plsc

_EMB_DIMS = (32, 32, 64, 32, 32)
_PACK_OFF = (0, 32, 64, 0, 32)
_TOTAL = 192
_MODEL = 768
_LG = 128
_BLK = 4096


def _gather_pack(x3, t0, t1, t2, t3, t4):
    n_groups = x3.shape[1]
    n_tok = n_groups * _LG
    tshapes = [(t.shape, t.dtype) for t in (t0, t1, t2, t3, t4)]
    info = plsc.get_sparse_core_info()
    nc = info.num_cores
    nw = nc * info.num_subcores
    g_per_w = n_groups // nw
    mesh = plsc.VectorSubcoreMesh(core_axis_name="c", subcore_axis_name="s")

    scratch = (
        [pltpu.VMEM((g_per_w, _LG), jnp.int32) for _ in range(5)]
        + [pltpu.VMEM((_LG, d), jnp.float32) for d in _EMB_DIMS]
        + [pltpu.VMEM((_LG, d), jnp.float32) for d in _EMB_DIMS]
        + [pltpu.VMEM((_LG, 64), jnp.float32)]
        + [pltpu.VMEM_SHARED(sh, dt) for sh, dt in tshapes]
        + [pltpu.SemaphoreType.DMA,
           pltpu.SemaphoreType.DMA,
           pltpu.SemaphoreType.DMA]
    )

    @functools.partial(
        pl.kernel,
        mesh=mesh,
        out_type=jax.ShapeDtypeStruct((2 * n_tok, _LG), jnp.float32),
        scratch_types=scratch,
        compiler_params=pltpu.CompilerParams(use_tc_tiling_on_sc=False),
    )
    def k(x_ref, r0, r1, r2, r3, r4, h_ref,
          i0, i1, i2, i3, i4,
          a0, a1, a2, a3, a4,
          b0, b1, b2, b3, b4,
          zbuf, s0, s1, s2, s3, s4,
          sg, ss0, ss1):
        tbls_hbm = (r0, r1, r2, r3, r4)
        tbls = (s0, s1, s2, s3, s4)
        idxs = (i0, i1, i2, i3, i4)
        rows = ((a0, a1, a2, a3, a4), (b0, b1, b2, b3, b4))
        ssems = (ss0, ss1)
        sid = lax.axis_index("s")
        wid = sid * nc + lax.axis_index("c")
        g0 = wid * g_per_w

        @pl.when(sid == 0)
        def _stage():
            for f in range(5):
                pltpu.sync_copy(tbls_hbm[f], tbls[f])

        def zrow(t, _):
            for j in range(4):
                zbuf[t, pl.ds(16 * j, 16)] = jnp.zeros((16,), jnp.float32)
            return _
        lax.fori_loop(0, _LG, zrow, 0)

        for f in range(5):
            pltpu.sync_copy(x_ref.at[f, pl.ds(g0, g_per_w)], idxs[f])
        plsc.subcore_barrier()

        pending = [None, None]
        for it in range(g_per_w):
            s = it % 2
            if pending[s] is not None:
                for cp in pending[s]:
                    cp.wait()
            gathers = [
                pltpu.async_copy(tbls[f].at[idxs[f].at[it]], rows[s][f], sg)
                for f in range(5)
            ]
            for cp in gathers:
                cp.wait()
            row_l = (g0 + it) * _LG
            row_r = n_tok + row_l
            base = (row_l, row_l, row_l, row_r, row_r)
            pending[s] = [
                pltpu.async_copy(
                    rows[s][f],
                    h_ref.at[pl.ds(base[f], _LG),
                             pl.ds(_PACK_OFF[f], _EMB_DIMS[f])],
                    ssems[s])
                for f in range(5)
            ]
            pending[s].append(
                pltpu.async_copy(
                    zbuf, h_ref.at[pl.ds(row_r, _LG), pl.ds(64, 64)],
                    ssems[s]))
        for s in range(2):
            if pending[s] is not None:
                for cp in pending[s]:
                    cp.wait()

    return k(x3, t0, t1, t2, t3, t4)


def _encode(h2, wpad, enc_b2, n_tok):
    nblk = n_tok // _BLK

    def body(hl_ref, hr_ref, wl_ref, wr_ref, b_ref, o_ref):
        o_ref[...] = (
            jnp.dot(hl_ref[...], wl_ref[...],
                    preferred_element_type=jnp.float32)
            + jnp.dot(hr_ref[...], wr_ref[...],
                      preferred_element_type=jnp.float32)
            + b_ref[...]
        )

    return pl.pallas_call(
        body,
        grid=(nblk,),
        in_specs=[
            pl.BlockSpec((_BLK, _LG), lambda i: (i, 0)),
            pl.BlockSpec((_BLK, _LG), lambda i, *, nb=nblk: (nb + i, 0)),
            pl.BlockSpec((_LG, _MODEL), lambda i: (0, 0)),
            pl.BlockSpec((_LG, _MODEL), lambda i: (1, 0)),
            pl.BlockSpec((1, _MODEL), lambda i: (0, 0)),
        ],
        out_specs=pl.BlockSpec((_BLK, _MODEL), lambda i: (i, 0)),
        out_shape=jax.ShapeDtypeStruct((n_tok, _MODEL), jnp.float32),
    )(h2, h2, wpad, wpad, enc_b2)


def kernel(x, table_0, table_1, table_2, table_3, table_4, enc_w, enc_b):
    b, s, f = x.shape
    n_tok = b * s
    xi = x.astype(jnp.int32)
    x3 = xi.reshape(n_tok // _LG, _LG, f).transpose(2, 0, 1)
    h2 = _gather_pack(x3, table_0, table_1, table_2, table_3, table_4)
    wpad = jnp.pad(enc_w, ((0, 2 * _LG - _TOTAL), (0, 0)))
    out = _encode(h2, wpad, enc_b.reshape(1, _MODEL), n_tok)
    return out.reshape(b, s, _MODEL)


# speedup vs baseline: 1.4299x; 1.0432x over previous
import functools

import jax
import jax.numpy as jnp
from jax import lax
from jax.experimental import pallas as pl
from jax.experimental.pallas import tpu as pltpu
from jax.experimental.pallas import tpu_sc as plsc

_EMB_DIMS = (32, 32, 64, 32, 32)
_PACK_OFF = (0, 32, 64, 0, 32)
_TOTAL = 192
_MODEL = 768
_LG = 128
_BLK = 4096


def _gather_pack(x3, t0, t1, t2, t3, t4):
    n_groups = x3.shape[1]
    n_tok = n_groups * _LG
    tshapes = [(t.shape, t.dtype) for t in (t0, t1, t2, t3, t4)]
    info = plsc.get_sparse_core_info()
    nc = info.num_cores
    nw = nc * info.num_subcores
    g_per_w = n_groups // nw
    mesh = plsc.VectorSubcoreMesh(core_axis_name="c", subcore_axis_name="s")

    scratch = (
        [pltpu.VMEM((g_per_w, _LG), jnp.int32) for _ in range(5)]
        + [pltpu.VMEM((_LG, d), jnp.float32) for d in _EMB_DIMS]
        + [pltpu.VMEM((_LG, d), jnp.float32) for d in _EMB_DIMS]
        + [pltpu.VMEM((_LG, 64), jnp.float32)]
        + [pltpu.VMEM_SHARED(sh, dt) for sh, dt in tshapes]
        + [pltpu.SemaphoreType.DMA,
           pltpu.SemaphoreType.DMA,
           pltpu.SemaphoreType.DMA]
    )

    @functools.partial(
        pl.kernel,
        mesh=mesh,
        out_type=jax.ShapeDtypeStruct((2 * n_tok, _LG), jnp.float32),
        scratch_types=scratch,
        compiler_params=pltpu.CompilerParams(use_tc_tiling_on_sc=False),
    )
    def k(x_ref, r0, r1, r2, r3, r4, h_ref,
          i0, i1, i2, i3, i4,
          a0, a1, a2, a3, a4,
          b0, b1, b2, b3, b4,
          zbuf, s0, s1, s2, s3, s4,
          sg, ss0, ss1):
        tbls_hbm = (r0, r1, r2, r3, r4)
        tbls = (s0, s1, s2, s3, s4)
        idxs = (i0, i1, i2, i3, i4)
        rows = ((a0, a1, a2, a3, a4), (b0, b1, b2, b3, b4))
        ssems = (ss0, ss1)
        sid = lax.axis_index("s")
        wid = sid * nc + lax.axis_index("c")
        g0 = wid * g_per_w

        stage = []
        for f in range(5):
            nr = tbls[f].shape[0] // 16
            stage.append(pltpu.async_copy(
                tbls_hbm[f].at[pl.ds(sid * nr, nr)],
                tbls[f].at[pl.ds(sid * nr, nr)], sg))

        def zrow(t, _):
            for j in range(4):
                zbuf[t, pl.ds(16 * j, 16)] = jnp.zeros((16,), jnp.float32)
            return _
        lax.fori_loop(0, _LG, zrow, 0)

        for f in range(5):
            pltpu.sync_copy(x_ref.at[f, pl.ds(g0, g_per_w)], idxs[f])
        for cp in stage:
            cp.wait()
        plsc.subcore_barrier()

        pending = [None, None]
        for it in range(g_per_w):
            s = it % 2
            if pending[s] is not None:
                for cp in pending[s]:
                    cp.wait()
            gathers = [
                pltpu.async_copy(tbls[f].at[idxs[f].at[it]], rows[s][f], sg)
                for f in range(5)
            ]
            for cp in gathers:
                cp.wait()
            row_l = (g0 + it) * _LG
            row_r = n_tok + row_l
            base = (row_l, row_l, row_l, row_r, row_r)
            pending[s] = [
                pltpu.async_copy(
                    rows[s][f],
                    h_ref.at[pl.ds(base[f], _LG),
                             pl.ds(_PACK_OFF[f], _EMB_DIMS[f])],
                    ssems[s])
                for f in range(5)
            ]
            pending[s].append(
                pltpu.async_copy(
                    zbuf, h_ref.at[pl.ds(row_r, _LG), pl.ds(64, 64)],
                    ssems[s]))
        for s in range(2):
            if pending[s] is not None:
                for cp in pending[s]:
                    cp.wait()

    return k(x3, t0, t1, t2, t3, t4)


def _encode(h2, wpad, enc_b2, n_tok):
    nblk = n_tok // _BLK

    def body(hl_ref, hr_ref, wl_ref, wr_ref, b_ref, o_ref):
        o_ref[...] = (
            jnp.dot(hl_ref[...], wl_ref[...],
                    preferred_element_type=jnp.float32)
            + jnp.dot(hr_ref[...], wr_ref[...],
                      preferred_element_type=jnp.float32)
            + b_ref[...]
        )

    return pl.pallas_call(
        body,
        grid=(nblk,),
        in_specs=[
            pl.BlockSpec((_BLK, _LG), lambda i: (i, 0)),
            pl.BlockSpec((_BLK, _LG), lambda i, *, nb=nblk: (nb + i, 0)),
            pl.BlockSpec((_LG, _MODEL), lambda i: (0, 0)),
            pl.BlockSpec((_LG, _MODEL), lambda i: (1, 0)),
            pl.BlockSpec((1, _MODEL), lambda i: (0, 0)),
        ],
        out_specs=pl.BlockSpec((_BLK, _MODEL), lambda i: (i, 0)),
        out_shape=jax.ShapeDtypeStruct((n_tok, _MODEL), jnp.float32),
    )(h2, h2, wpad, wpad, enc_b2)


def kernel(x, table_0, table_1, table_2, table_3, table_4, enc_w, enc_b):
    b, s, f = x.shape
    n_tok = b * s
    xi = x.astype(jnp.int32)
    x3 = xi.reshape(n_tok // _LG, _LG, f).transpose(2, 0, 1)
    h2 = _gather_pack(x3, table_0, table_1, table_2, table_3, table_4)
    wpad = jnp.pad(enc_w, ((0, 2 * _LG - _TOTAL), (0, 0)))
    out = _encode(h2, wpad, enc_b.reshape(1, _MODEL), n_tok)
    return out.reshape(b, s, _MODEL)
